# Initial kernel scaffold; baseline (speedup 1.0000x reference)
import jax
import jax.numpy as jnp
from jax.experimental import pallas as pl


def kernel(user_emb, item_emb, adj_indices, adj_values):
    raise NotImplementedError("write your pallas kernel here")


# speedup vs baseline: 10.1879x; 10.1879x over previous
import functools

import jax
import jax.numpy as jnp
from jax import lax
from jax.experimental import pallas as pl
from jax.experimental.pallas import tpu as pltpu
from jax.experimental.pallas import tpu_sc as plsc

N_USERS = 50000
N_ITEMS = 100000
N_NODES = N_USERS + N_ITEMS
NNZ = 3200000
EMB = 64
N_LAYERS = 3

NC = 2
NS = 16
BLK = 128
NBLKS = NNZ // BLK
R = N_NODES // (3 * NC)
R_PAD = R + 8
DUMP = R
QUOTA = 1600
OB = 160
N_PASSES = 3


def _spmm_body(first, ego_hbm, mean_in_hbm, cols_hbm, rows_hbm, vals_hbm,
               spans_hbm, ego_out_hbm, mean_out_hbm,
               span_v, colbuf, rowbuf, valbuf, idxbuf,
               gbuf, obuf, mbuf, acc_sh, sem):
    c = lax.axis_index("c")
    s = lax.axis_index("s")
    zeros16 = jnp.zeros((16,), jnp.float32)
    mean_in_scale = 0.25 if first else 1.0
    tile_start = s * QUOTA

    @pl.loop(0, N_PASSES)
    def _pass(p):
        k = p * NC + c
        row_base = k * R
        pltpu.sync_copy(spans_hbm.at[k], span_v)
        sv = span_v[...]
        b_lo = sv[0]
        b_hi = sv[1]

        @pl.loop(0, OB)
        def _zrow(i):
            for d in range(4):
                obuf[i, pl.ds(d * 16, 16)] = zeros16

        @pl.loop(0, QUOTA // OB)
        def _zchunk(kk):
            st = jnp.minimum(tile_start + kk * OB, R - OB)
            pltpu.sync_copy(obuf, acc_sh.at[pl.ds(st, OB)])

        plsc.subcore_barrier()

        nb = b_hi - b_lo
        per = (nb + NS - 1) // NS
        t_lo = b_lo + s * per
        t_hi = jnp.minimum(b_hi, t_lo + per)

        @pl.loop(0, jnp.maximum(t_hi - t_lo, 0))
        def _blk(i):
            blk = t_lo + i
            pltpu.sync_copy(cols_hbm.at[blk], colbuf)
            pltpu.sync_copy(rows_hbm.at[blk], rowbuf)
            pltpu.sync_copy(vals_hbm.at[blk], valbuf)
            pltpu.async_copy(ego_hbm.at[colbuf], gbuf, sem).wait()

            @pl.loop(0, BLK // 16)
            def _grp(j8):
                rv = rowbuf[pl.ds(j8 * 16, 16)]
                rl = rv - row_base
                m = (rl >= 0) & (rl < R)
                idxbuf[pl.ds(j8 * 16, 16)] = jnp.where(m, rl, DUMP)
                vv = valbuf[pl.ds(j8 * 16, 16)]
                for l in range(16):
                    j = j8 * 16 + l
                    bv = jnp.full((16,), vv[l], jnp.float32)
                    for d in range(4):
                        sl = pl.ds(d * 16, 16)
                        gbuf[j, sl] = gbuf[j, sl] * bv

            pltpu.sync_copy(gbuf, acc_sh.at[idxbuf], add=True)

        plsc.subcore_barrier()

        @pl.loop(0, QUOTA // OB)
        def _ochunk(kk):
            lst = jnp.minimum(tile_start + kk * OB, R - OB)
            gst = row_base + lst
            pltpu.sync_copy(acc_sh.at[pl.ds(lst, OB)], obuf)
            pltpu.sync_copy(mean_in_hbm.at[pl.ds(gst, OB)], mbuf)

            @pl.loop(0, OB)
            def _row(i):
                for d in range(4):
                    sl = pl.ds(d * 16, 16)
                    mbuf[i, sl] = mbuf[i, sl] * mean_in_scale + obuf[i, sl] * 0.25

            pltpu.sync_copy(obuf, ego_out_hbm.at[pl.ds(gst, OB)])
            pltpu.sync_copy(mbuf, mean_out_hbm.at[pl.ds(gst, OB)])

        plsc.subcore_barrier()


@functools.lru_cache(maxsize=None)
def _make_spmm(first):
    mesh = plsc.VectorSubcoreMesh(core_axis_name="c", subcore_axis_name="s",
                                  num_cores=NC, num_subcores=NS)
    f32 = jnp.float32
    return pl.kernel(
        functools.partial(_spmm_body, first),
        out_type=(jax.ShapeDtypeStruct((N_NODES, EMB), f32),
                  jax.ShapeDtypeStruct((N_NODES, EMB), f32)),
        mesh=mesh,
        compiler_params=pltpu.CompilerParams(use_tc_tiling_on_sc=False),
        scratch_types=[
            pltpu.VMEM((16,), jnp.int32),
            pltpu.VMEM((BLK,), jnp.int32),
            pltpu.VMEM((BLK,), jnp.int32),
            pltpu.VMEM((BLK,), f32),
            pltpu.VMEM((BLK,), jnp.int32),
            pltpu.VMEM((BLK, EMB), f32),
            pltpu.VMEM((OB, EMB), f32),
            pltpu.VMEM((OB, EMB), f32),
            pltpu.VMEM_SHARED((R_PAD, EMB), f32),
            pltpu.SemaphoreType.DMA,
        ],
        name="lgcn_spmm_layer",
    )


def kernel(user_emb, item_emb, adj_indices, adj_values):
    rows = adj_indices[0]
    cols = adj_indices[1]
    ego0 = jnp.concatenate([user_emb, item_emb], axis=0)

    rows_s, cols_s, vals_s = lax.sort((rows, cols, adj_values), num_keys=1)
    interior = jnp.arange(1, N_PASSES * NC, dtype=jnp.int32) * R
    e = jnp.searchsorted(rows_s, interior).astype(jnp.int32)
    lo_e = jnp.concatenate([jnp.zeros((1,), jnp.int32), e])
    hi_e = jnp.concatenate([e, jnp.full((1,), NNZ, jnp.int32)])
    b_lo = lo_e // BLK
    b_hi = (hi_e + BLK - 1) // BLK
    spans = jnp.zeros((N_PASSES * NC, 16), jnp.int32)
    spans = spans.at[:, 0].set(b_lo).at[:, 1].set(b_hi)

    cols2d = cols_s.reshape(NBLKS, BLK)
    rows2d = rows_s.reshape(NBLKS, BLK)
    vals2d = vals_s.reshape(NBLKS, BLK)

    ego = ego0
    mean = ego0
    for layer in range(N_LAYERS):
        ego, mean = _make_spmm(layer == 0)(
            ego, mean, cols2d, rows2d, vals2d, spans)
    return mean[:N_USERS], mean[N_USERS:]


# speedup vs baseline: 13.9104x; 1.3654x over previous
import functools

import jax
import jax.numpy as jnp
from jax import lax
from jax.experimental import pallas as pl
from jax.experimental.pallas import tpu as pltpu
from jax.experimental.pallas import tpu_sc as plsc

N_USERS = 50000
N_ITEMS = 100000
N_NODES = N_USERS + N_ITEMS
NNZ = 3200000
EMB = 64
N_LAYERS = 3

NC = 2
NS = 16
BLK = 128
NBLKS = NNZ // BLK
CH = 2
N_PASSES = 4
R = N_NODES // (N_PASSES * NC)
R_PAD = R + 8
DUMP = R
QUOTA = 1248
OB = 96


def _spmm_body(first, ego_hbm, mean_in_hbm, meta_hbm, vals_hbm, spans_hbm,
               ego_out_hbm, mean_out_hbm,
               span_v, mc0, mc1, mv0, mv1, gb0, gb1, ix0, ix1,
               obuf, mbuf, acc_sh, gsem0, gsem1, ssem):
    c = lax.axis_index("c")
    s = lax.axis_index("s")
    zeros16 = jnp.zeros((16,), jnp.float32)
    mean_in_scale = 0.25 if first else 1.0
    tile_start = s * QUOTA
    mcs = (mc0, mc1)
    mvs = (mv0, mv1)
    gbs = (gb0, gb1)
    ixs = (ix0, ix1)
    gsems = (gsem0, gsem1)

    def fire_gathers(par, cs, t_hi):
        for j in range(CH):
            @pl.when(cs + j < t_hi)
            def _g():
                pltpu.async_copy(ego_hbm.at[mcs[par].at[j, 0]],
                                 gbs[par].at[j], gsems[par])

    def drain(sem, n_rows):
        pltpu.make_async_copy(ego_hbm.at[pl.ds(0, n_rows)],
                              gbs[0].at[0] if n_rows == BLK else obuf,
                              sem).wait()

    def compute_block(par, j, row_base):
        mc = mcs[par]
        gb = gbs[par]
        ix = ixs[par]

        @pl.loop(0, BLK // 16)
        def _grp(j8):
            sl16 = pl.ds(j8 * 16, 16)
            rv = mc[j, 1, sl16]
            rl = rv - row_base
            m = (rl >= 0) & (rl < R)
            ix[j, sl16] = jnp.where(m, rl, DUMP)
            vv = mvs[par][j, sl16]
            for l in range(16):
                e = j8 * 16 + l
                bv = jnp.full((16,), vv[l], jnp.float32)
                for d in range(4):
                    sl = pl.ds(d * 16, 16)
                    gb[j, e, sl] = gb[j, e, sl] * bv

    @pl.loop(0, N_PASSES)
    def _pass(p):
        k = p * NC + c
        row_base = k * R
        pltpu.sync_copy(spans_hbm.at[k], span_v)
        sv = span_v[...]
        b_lo = sv[0]
        b_hi = sv[1]

        @pl.loop(0, OB)
        def _zrow(i):
            for d in range(4):
                obuf[i, pl.ds(d * 16, 16)] = zeros16

        for kk in range(QUOTA // OB):
            st = jnp.minimum(tile_start + kk * OB, R - OB)
            pltpu.async_copy(obuf, acc_sh.at[pl.ds(st, OB)], ssem)
        for kk in range(QUOTA // OB):
            drain(ssem, OB)

        plsc.subcore_barrier()

        nb = b_hi - b_lo
        per = (nb + NS - 1) // NS
        t_lo = b_lo + s * per
        t_hi = jnp.minimum(b_hi, t_lo + per)
        nt = jnp.maximum(t_hi - t_lo, 0)
        n_ch = (nt + CH - 1) // CH

        @pl.when(nt > 0)
        def _prime():
            pltpu.sync_copy(meta_hbm.at[pl.ds(t_lo, CH)], mc0)
            pltpu.sync_copy(vals_hbm.at[pl.ds(t_lo, CH)], mv0)
            fire_gathers(0, t_lo, t_hi)

        @pl.loop(0, (n_ch + 1) // 2)
        def _gpair(gp):
            for par in range(2):
                g = gp * 2 + par

                @pl.when(g < n_ch)
                def _chunk():
                    cs = t_lo + g * CH

                    @pl.when(g + 1 < n_ch)
                    def _pref():
                        ncs = cs + CH
                        pltpu.sync_copy(meta_hbm.at[pl.ds(ncs, CH)],
                                        mcs[1 - par])
                        pltpu.sync_copy(vals_hbm.at[pl.ds(ncs, CH)],
                                        mvs[1 - par])
                        fire_gathers(1 - par, ncs, t_hi)

                    for j in range(CH):
                        @pl.when(cs + j < t_hi)
                        def _d():
                            drain(gsems[par], BLK)

                    for j in range(CH):
                        @pl.when(cs + j < t_hi)
                        def _c(j=j):
                            compute_block(par, j, row_base)
                            pltpu.async_copy(gbs[par].at[j],
                                             acc_sh.at[ixs[par].at[j]],
                                             ssem, add=True)

                    for j in range(CH):
                        @pl.when(cs + j < t_hi)
                        def _ds():
                            drain(ssem, BLK)

        plsc.subcore_barrier()

        @pl.loop(0, QUOTA // OB)
        def _ochunk(kk):
            lst = jnp.minimum(tile_start + kk * OB, R - OB)
            gst = row_base + lst
            pltpu.async_copy(acc_sh.at[pl.ds(lst, OB)], obuf, gsem0)
            pltpu.async_copy(mean_in_hbm.at[pl.ds(gst, OB)], mbuf, gsem1)
            pltpu.make_async_copy(ego_hbm.at[pl.ds(0, OB)], obuf,
                                  gsem0).wait()
            pltpu.make_async_copy(ego_hbm.at[pl.ds(0, OB)], mbuf,
                                  gsem1).wait()

            @pl.loop(0, OB)
            def _row(i):
                for d in range(4):
                    sl = pl.ds(d * 16, 16)
                    mbuf[i, sl] = mbuf[i, sl] * mean_in_scale + obuf[i, sl] * 0.25

            pltpu.sync_copy(obuf, ego_out_hbm.at[pl.ds(gst, OB)])
            pltpu.sync_copy(mbuf, mean_out_hbm.at[pl.ds(gst, OB)])

        plsc.subcore_barrier()


@functools.lru_cache(maxsize=None)
def _make_spmm(first):
    mesh = plsc.VectorSubcoreMesh(core_axis_name="c", subcore_axis_name="s",
                                  num_cores=NC, num_subcores=NS)
    f32 = jnp.float32
    return pl.kernel(
        functools.partial(_spmm_body, first),
        out_type=(jax.ShapeDtypeStruct((N_NODES, EMB), f32),
                  jax.ShapeDtypeStruct((N_NODES, EMB), f32)),
        mesh=mesh,
        compiler_params=pltpu.CompilerParams(use_tc_tiling_on_sc=False),
        scratch_types=[
            pltpu.VMEM((16,), jnp.int32),
            pltpu.VMEM((CH, 2, BLK), jnp.int32),
            pltpu.VMEM((CH, 2, BLK), jnp.int32),
            pltpu.VMEM((CH, BLK), f32),
            pltpu.VMEM((CH, BLK), f32),
            pltpu.VMEM((CH, BLK, EMB), f32),
            pltpu.VMEM((CH, BLK, EMB), f32),
            pltpu.VMEM((CH, BLK), jnp.int32),
            pltpu.VMEM((CH, BLK), jnp.int32),
            pltpu.VMEM((OB, EMB), f32),
            pltpu.VMEM((OB, EMB), f32),
            pltpu.VMEM_SHARED((R_PAD, EMB), f32),
            pltpu.SemaphoreType.DMA,
            pltpu.SemaphoreType.DMA,
            pltpu.SemaphoreType.DMA,
        ],
        name="lgcn_spmm_layer",
    )


def kernel(user_emb, item_emb, adj_indices, adj_values):
    rows = adj_indices[0]
    cols = adj_indices[1]
    ego0 = jnp.concatenate([user_emb, item_emb], axis=0)

    rows_s, cols_s, vals_s = lax.sort((rows, cols, adj_values), num_keys=1)
    interior = jnp.arange(1, N_PASSES * NC, dtype=jnp.int32) * R
    e = jnp.searchsorted(rows_s, interior).astype(jnp.int32)
    lo_e = jnp.concatenate([jnp.zeros((1,), jnp.int32), e])
    hi_e = jnp.concatenate([e, jnp.full((1,), NNZ, jnp.int32)])
    b_lo = lo_e // BLK
    b_hi = (hi_e + BLK - 1) // BLK
    spans = jnp.zeros((N_PASSES * NC, 16), jnp.int32)
    spans = spans.at[:, 0].set(b_lo).at[:, 1].set(b_hi)

    meta = jnp.stack([cols_s.reshape(NBLKS, BLK),
                      rows_s.reshape(NBLKS, BLK)], axis=1)
    meta = jnp.pad(meta, ((0, 2 * CH), (0, 0), (0, 0)))
    vals2d = jnp.pad(vals_s.reshape(NBLKS, BLK), ((0, 2 * CH), (0, 0)))

    ego = ego0
    mean = ego0
    for layer in range(N_LAYERS):
        ego, mean = _make_spmm(layer == 0)(ego, mean, meta, vals2d, spans)
    return mean[:N_USERS], mean[N_USERS:]


# speedup vs baseline: 15.3985x; 1.1070x over previous
import functools

import jax
import jax.numpy as jnp
from jax import lax
from jax.experimental import pallas as pl
from jax.experimental.pallas import tpu as pltpu
from jax.experimental.pallas import tpu_sc as plsc

N_USERS = 50000
N_ITEMS = 100000
N_NODES = N_USERS + N_ITEMS
NNZ = 3200000
EMB = 64
N_LAYERS = 3

NC = 2
NS = 16
BLK = 128
NBLKS = NNZ // BLK
CH = 2
N_PASSES = 4
R = N_NODES // (N_PASSES * NC)
R_PAD = R + 8
DUMP = R
QUOTA = 1248
OB = 96


def _spmm_body(first, ego_hbm, mean_in_hbm, meta_hbm, vals_hbm, spans_hbm,
               ego_out_hbm, mean_out_hbm,
               span_v, mc0, mc1, mv0, mv1, gb0, gb1, ix0, ix1,
               obuf, mbuf, acc_sh, gsem0, gsem1, ssem):
    c = lax.axis_index("c")
    s = lax.axis_index("s")
    zeros16 = jnp.zeros((16,), jnp.float32)
    mean_in_scale = 0.25 if first else 1.0
    tile_start = s * QUOTA
    mcs = (mc0, mc1)
    mvs = (mv0, mv1)
    gbs = (gb0, gb1)
    ixs = (ix0, ix1)
    gsems = (gsem0, gsem1)

    def fire_gathers(par, cs, t_hi):
        for j in range(CH):
            @pl.when(cs + j < t_hi)
            def _g():
                pltpu.async_copy(ego_hbm.at[mcs[par].at[j, 0]],
                                 gbs[par].at[j], gsems[par])

    def drain(sem, n_rows):
        pltpu.make_async_copy(ego_hbm.at[pl.ds(0, n_rows)],
                              gbs[0].at[0] if n_rows == BLK else obuf,
                              sem).wait()

    def compute_block(par, j, row_base):
        mc = mcs[par]
        gb = gbs[par]
        ix = ixs[par]

        @pl.loop(0, BLK // 16)
        def _grp(j8):
            sl16 = pl.ds(j8 * 16, 16)
            rv = mc[j, 1, sl16]
            rl = rv - row_base
            m = (rl >= 0) & (rl < R)
            ix[j, sl16] = jnp.where(m, rl, DUMP)
            vv = mvs[par][j, sl16]
            for l in range(16):
                e = j8 * 16 + l
                bv = jnp.full((16,), vv[l], jnp.float32)
                for d in range(4):
                    sl = pl.ds(d * 16, 16)
                    gb[j, e, sl] = gb[j, e, sl] * bv

    @pl.loop(0, N_PASSES)
    def _pass(p):
        k = p * NC + c
        row_base = k * R
        pltpu.sync_copy(spans_hbm.at[k], span_v)
        sv = span_v[...]
        b_lo = sv[0]
        b_hi = sv[1]

        @pl.loop(0, OB)
        def _zrow(i):
            for d in range(4):
                obuf[i, pl.ds(d * 16, 16)] = zeros16

        for kk in range(QUOTA // OB):
            st = jnp.minimum(tile_start + kk * OB, R - OB)
            pltpu.async_copy(obuf, acc_sh.at[pl.ds(st, OB)], ssem)
        for kk in range(QUOTA // OB):
            drain(ssem, OB)

        plsc.subcore_barrier()

        nb = b_hi - b_lo
        per = (nb + NS - 1) // NS
        t_lo = b_lo + s * per
        t_hi = jnp.minimum(b_hi, t_lo + per)
        nt = jnp.maximum(t_hi - t_lo, 0)
        n_ch = (nt + CH - 1) // CH

        @pl.when(nt > 0)
        def _prime():
            pltpu.sync_copy(meta_hbm.at[pl.ds(t_lo, CH)], mc0)
            pltpu.sync_copy(vals_hbm.at[pl.ds(t_lo, CH)], mv0)
            fire_gathers(0, t_lo, t_hi)

        @pl.loop(0, (n_ch + 1) // 2)
        def _gpair(gp):
            for par in range(2):
                g = gp * 2 + par

                @pl.when(g < n_ch)
                def _chunk():
                    cs = t_lo + g * CH

                    @pl.when(g + 1 < n_ch)
                    def _pref():
                        ncs = cs + CH
                        pltpu.sync_copy(meta_hbm.at[pl.ds(ncs, CH)],
                                        mcs[1 - par])
                        pltpu.sync_copy(vals_hbm.at[pl.ds(ncs, CH)],
                                        mvs[1 - par])
                        fire_gathers(1 - par, ncs, t_hi)

                    for j in range(CH):
                        @pl.when(cs + j < t_hi)
                        def _d():
                            drain(gsems[par], BLK)

                    for j in range(CH):
                        @pl.when(cs + j < t_hi)
                        def _c(j=j):
                            compute_block(par, j, row_base)
                            pltpu.async_copy(gbs[par].at[j],
                                             acc_sh.at[ixs[par].at[j]],
                                             ssem, add=True)

                    for j in range(CH):
                        @pl.when(cs + j < t_hi)
                        def _ds():
                            drain(ssem, BLK)

        plsc.subcore_barrier()

        @pl.loop(0, QUOTA // OB)
        def _ochunk(kk):
            lst = jnp.minimum(tile_start + kk * OB, R - OB)
            gst = row_base + lst
            pltpu.async_copy(acc_sh.at[pl.ds(lst, OB)], obuf, gsem0)
            pltpu.async_copy(mean_in_hbm.at[pl.ds(gst, OB)], mbuf, gsem1)
            pltpu.make_async_copy(ego_hbm.at[pl.ds(0, OB)], obuf,
                                  gsem0).wait()
            pltpu.make_async_copy(ego_hbm.at[pl.ds(0, OB)], mbuf,
                                  gsem1).wait()

            @pl.loop(0, OB)
            def _row(i):
                for d in range(4):
                    sl = pl.ds(d * 16, 16)
                    mbuf[i, sl] = mbuf[i, sl] * mean_in_scale + obuf[i, sl] * 0.25

            pltpu.sync_copy(obuf, ego_out_hbm.at[pl.ds(gst, OB)])
            pltpu.sync_copy(mbuf, mean_out_hbm.at[pl.ds(gst, OB)])

        plsc.subcore_barrier()


@functools.lru_cache(maxsize=None)
def _make_spmm(first):
    mesh = plsc.VectorSubcoreMesh(core_axis_name="c", subcore_axis_name="s",
                                  num_cores=NC, num_subcores=NS)
    f32 = jnp.float32
    return pl.kernel(
        functools.partial(_spmm_body, first),
        out_type=(jax.ShapeDtypeStruct((N_NODES, EMB), f32),
                  jax.ShapeDtypeStruct((N_NODES, EMB), f32)),
        mesh=mesh,
        compiler_params=pltpu.CompilerParams(use_tc_tiling_on_sc=False),
        scratch_types=[
            pltpu.VMEM((16,), jnp.int32),
            pltpu.VMEM((CH, 2, BLK), jnp.int32),
            pltpu.VMEM((CH, 2, BLK), jnp.int32),
            pltpu.VMEM((CH, BLK), f32),
            pltpu.VMEM((CH, BLK), f32),
            pltpu.VMEM((CH, BLK, EMB), f32),
            pltpu.VMEM((CH, BLK, EMB), f32),
            pltpu.VMEM((CH, BLK), jnp.int32),
            pltpu.VMEM((CH, BLK), jnp.int32),
            pltpu.VMEM((OB, EMB), f32),
            pltpu.VMEM((OB, EMB), f32),
            pltpu.VMEM_SHARED((R_PAD, EMB), f32),
            pltpu.SemaphoreType.DMA,
            pltpu.SemaphoreType.DMA,
            pltpu.SemaphoreType.DMA,
        ],
        name="lgcn_spmm_layer",
    )


def kernel(user_emb, item_emb, adj_indices, adj_values):
    rows = adj_indices[0]
    cols = adj_indices[1]
    ego0 = jnp.concatenate([user_emb, item_emb], axis=0)

    perm = jnp.argsort(rows)
    rows_s = rows[perm]
    cols_s = cols[perm]
    vals_s = adj_values[perm]
    interior = jnp.arange(1, N_PASSES * NC, dtype=jnp.int32) * R
    e = jnp.searchsorted(rows_s, interior).astype(jnp.int32)
    lo_e = jnp.concatenate([jnp.zeros((1,), jnp.int32), e])
    hi_e = jnp.concatenate([e, jnp.full((1,), NNZ, jnp.int32)])
    b_lo = lo_e // BLK
    b_hi = (hi_e + BLK - 1) // BLK
    spans = jnp.zeros((N_PASSES * NC, 16), jnp.int32)
    spans = spans.at[:, 0].set(b_lo).at[:, 1].set(b_hi)

    meta = jnp.stack([cols_s.reshape(NBLKS, BLK),
                      rows_s.reshape(NBLKS, BLK)], axis=1)
    meta = jnp.pad(meta, ((0, 2 * CH), (0, 0), (0, 0)))
    vals2d = jnp.pad(vals_s.reshape(NBLKS, BLK), ((0, 2 * CH), (0, 0)))

    ego = ego0
    mean = ego0
    for layer in range(N_LAYERS):
        ego, mean = _make_spmm(layer == 0)(ego, mean, meta, vals2d, spans)
    return mean[:N_USERS], mean[N_USERS:]
